# Initial kernel scaffold; baseline (speedup 1.0000x reference)
import jax
import jax.numpy as jnp
from jax.experimental import pallas as pl


def kernel(inputs, embedding):
    raise NotImplementedError("write your pallas kernel here")


# speedup vs baseline: 5.5259x; 5.5259x over previous
import jax
import jax.numpy as jnp
from jax.experimental import pallas as pl
from jax.experimental.pallas import tpu as pltpu
from jax.experimental.pallas import tpu_sc as plsc

NUM_CODES = 8192
EMB_DIM = 256
COMMIT = 0.6
B1, B2 = 2736, 5472
TB = 128
GW = 128
ETB = 256


def _dist_argmin_kernel(x_ref, x2_ref, e_ref, idx_ref, cnt_ref):
    x = x_ref[...]
    e = e_ref[...]
    x2 = x2_ref[...]
    mm = jax.lax.dot_general(x, e, (((1,), (1,)), ((), ())),
                             preferred_element_type=jnp.float32)
    d = x2 - 2.0 * mm
    j = jax.lax.broadcasted_iota(jnp.int32, d.shape, 1)
    inf = jnp.float32(jnp.inf)
    acc_v = None
    acc_i = None
    for a, b in ((0, B1), (B1, B2), (B2, NUM_CODES)):
        dm = jnp.where((j >= a) & (j < b), d, inf)
        v = jnp.min(dm, axis=1, keepdims=True)
        im = jnp.min(jnp.where(dm == v, j, jnp.int32(NUM_CODES)),
                     axis=1, keepdims=True)
        if acc_v is None:
            acc_v, acc_i = v, im
        else:
            keep = acc_v <= v
            acc_i = jnp.where(keep, acc_i, im)
            acc_v = jnp.where(keep, acc_v, v)
        if b != NUM_CODES:
            acc_v = acc_v.astype(jnp.bfloat16).astype(jnp.float32)
    idx_ref[0, 0, :] = acc_i[:, 0]

    onehot = jnp.where(j == acc_i, 1.0, 0.0).astype(jnp.float32)
    step = pl.program_id(0)

    @pl.when(step == 0)
    def _():
        cnt_ref[...] = jnp.zeros_like(cnt_ref)

    cnt_ref[0, :] += jnp.sum(onehot, axis=0)


def _sc_gather(eb, idx2d, n):
    mesh = plsc.VectorSubcoreMesh(core_axis_name="c", subcore_axis_name="s")

    @pl.kernel(out_type=jax.ShapeDtypeStruct((n, EMB_DIM), eb.dtype),
               mesh=mesh)
    def _gather_kernel(e_hbm, i_hbm, o_hbm):
        def body(i_vmem, o_vmem):
            pltpu.sync_copy(e_hbm.at[i_vmem.at[0]], o_vmem)

        pltpu.emit_pipeline(
            body,
            grid=(n // GW,),
            in_specs=[pl.BlockSpec((1, GW), index_map=lambda i: (0, i))],
            out_specs=[pl.BlockSpec((GW, EMB_DIM), index_map=lambda i: (i, 0))],
            core_axis_name=("c", "s"),
            dimension_semantics=(pltpu.PARALLEL,),
        )(i_hbm, o_hbm)

    return _gather_kernel(eb, idx2d)


def _epilogue_kernel(q_ref, x_ref, cnt_ref, out_ref, loss_ref, ppl_ref, s_ref):
    q = q_ref[...]
    x = x_ref[...]
    diff = q - x
    qst = x + diff
    out_ref[0] = qst.T

    step = pl.program_id(0)
    nstep = pl.num_programs(0)

    @pl.when(step == 0)
    def _():
        s_ref[0, 0] = 0.0

    s_ref[0, 0] += jnp.sum(diff * diff)

    @pl.when(step == nstep - 1)
    def _():
        m = s_ref[0, 0] * jnp.float32(1.0 / (16384.0 * 256.0))
        loss_ref[...] = jnp.broadcast_to(m + jnp.float32(COMMIT) * m, (1, 1))
        p = cnt_ref[0, :] * jnp.float32(1.0 / 16384.0)
        t = p * jnp.log(p + 1e-10)
        ppl_ref[...] = jnp.broadcast_to(jnp.exp(-jnp.sum(t)), (1, 1))


def kernel(inputs, embedding):
    b, c, h, w = inputs.shape
    xt = jnp.transpose(inputs, (0, 2, 3, 1))
    flat = xt.reshape(-1, c)
    n = flat.shape[0]
    x2 = jnp.sum(xt * xt, axis=-1).reshape(-1, 1)

    idx_blocks, counts = pl.pallas_call(
        _dist_argmin_kernel,
        grid=(n // TB,),
        in_specs=[pl.BlockSpec((TB, EMB_DIM), lambda i: (i, 0)),
                  pl.BlockSpec((TB, 1), lambda i: (i, 0)),
                  pl.BlockSpec((NUM_CODES, EMB_DIM), lambda i: (0, 0))],
        out_specs=[pl.BlockSpec((1, 1, TB), lambda i: (i, 0, 0)),
                   pl.BlockSpec((1, NUM_CODES), lambda i: (0, 0))],
        out_shape=[jax.ShapeDtypeStruct((n // TB, 1, TB), jnp.int32),
                   jax.ShapeDtypeStruct((1, NUM_CODES), jnp.float32)],
    )(flat, x2, embedding)
    idx = idx_blocks.reshape(-1)

    eb = embedding.astype(jnp.bfloat16).astype(jnp.float32)
    q = _sc_gather(eb, idx.reshape(1, n), n)

    qst_t, loss, ppl = pl.pallas_call(
        _epilogue_kernel,
        grid=(n // ETB,),
        in_specs=[pl.BlockSpec((ETB, EMB_DIM), lambda i: (i, 0)),
                  pl.BlockSpec((ETB, EMB_DIM), lambda i: (i, 0)),
                  pl.BlockSpec((1, NUM_CODES), lambda i: (0, 0))],
        out_specs=[pl.BlockSpec((1, EMB_DIM, ETB), lambda i: (i, 0, 0)),
                   pl.BlockSpec((1, 1), lambda i: (0, 0)),
                   pl.BlockSpec((1, 1), lambda i: (0, 0))],
        out_shape=[jax.ShapeDtypeStruct((n // ETB, EMB_DIM, ETB), jnp.float32),
                   jax.ShapeDtypeStruct((1, 1), jnp.float32),
                   jax.ShapeDtypeStruct((1, 1), jnp.float32)],
        scratch_shapes=[pltpu.SMEM((1, 1), jnp.float32)],
    )(q, flat, counts)

    quantized_out = (qst_t.reshape(b, h * w // ETB, c, ETB)
                     .transpose(0, 2, 1, 3)
                     .reshape(b, c, h, w))
    encoding_indices = idx.reshape(b, h * w)
    return (loss[0, 0], quantized_out, ppl[0, 0], encoding_indices)
